# Initial kernel scaffold; baseline (speedup 1.0000x reference)
import jax
import jax.numpy as jnp
from jax.experimental import pallas as pl


def kernel(x, edge_index, edge_weight, W1z, b1z, W1r, b1r, W1h, b1h, W2z, b2z, W2r, b2r, W2h, b2h, lin_W, lin_b):
    raise NotImplementedError("write your pallas kernel here")


# speedup vs baseline: 10.5235x; 10.5235x over previous
import functools

import jax
import jax.numpy as jnp
from jax import lax
from jax.experimental import pallas as pl
from jax.experimental.pallas import tpu as pltpu
from jax.experimental.pallas import tpu_sc as plsc

N = 10000
NP = 10240
E = 320000
D = 128
H1 = 50
H2 = 20
WP = 128
S1 = 112
S2 = 48
NC = 2
NS = 16
NW = NC * NS
RPT = NP // NS
CK = 128
BR = 1024

_mesh = plsc.VectorSubcoreMesh(core_axis_name="c", subcore_axis_name="s")
_CP = pltpu.CompilerParams(needs_layout_passes=False)


def _zero_vec_ref(ref, n):
    def body(i, _):
        ref[pl.ds(i * 16, 16)] = jnp.zeros((16,), jnp.float32)
        return 0
    lax.fori_loop(0, n // 16, body, 0)


def _zero_rows_ref(ref, nrows, width):
    def body(i, _):
        for j in range(width // 16):
            ref[i, pl.ds(j * 16, 16)] = jnp.zeros((16,), jnp.float32)
        return 0
    lax.fori_loop(0, nrows, body, 0)


_DCHUNK = 400

@functools.partial(
    pl.kernel,
    out_type=(jax.ShapeDtypeStruct((NW * NP,), jnp.float32),
              jax.ShapeDtypeStruct((NW * NP,), jnp.float32)),
    mesh=_mesh,
    compiler_params=_CP,
    scratch_types=[
        pltpu.VMEM((_DCHUNK,), jnp.int32),
        pltpu.VMEM((_DCHUNK,), jnp.int32),
        pltpu.VMEM((_DCHUNK,), jnp.float32),
        pltpu.VMEM((NP,), jnp.float32),
        pltpu.VMEM((NP,), jnp.float32),
    ],
)
def _deg_kernel(row_hbm, col_hbm, ew_hbm, out_o_hbm, out_i_hbm,
                rowb, colb, ewb, acc_o, acc_i):
    cid = lax.axis_index("c")
    sid = lax.axis_index("s")
    wid = cid * NS + sid
    _zero_vec_ref(acc_o, NP)
    _zero_vec_ref(acc_i, NP)
    e_per_w = E // NW

    def chunk(ck, _):
        base = wid * e_per_w + ck * _DCHUNK
        pltpu.sync_copy(row_hbm.at[pl.ds(base, _DCHUNK)], rowb)
        pltpu.sync_copy(col_hbm.at[pl.ds(base, _DCHUNK)], colb)
        pltpu.sync_copy(ew_hbm.at[pl.ds(base, _DCHUNK)], ewb)
        for g in range(_DCHUNK // 16):
            sl = pl.ds(g * 16, 16)
            w16 = ewb[sl]
            plsc.addupdate_scatter(acc_o, [rowb[sl]], w16)
            plsc.addupdate_scatter(acc_i, [colb[sl]], w16)
        return 0

    lax.fori_loop(0, e_per_w // _DCHUNK, chunk, 0)
    pltpu.sync_copy(acc_o, out_o_hbm.at[pl.ds(wid * NP, NP)])
    pltpu.sync_copy(acc_i, out_i_hbm.at[pl.ds(wid * NP, NP)])


def _recip_body(dego_ref, degi_ref, reco_ref, reci_ref):
    dego = jnp.sum(dego_ref[...], axis=0)
    degi = jnp.sum(degi_ref[...], axis=0)
    reco_ref[...] = jnp.where(dego == 0.0, 1.0, 1.0 / dego)
    reci_ref[...] = jnp.where(degi == 0.0, 1.0, 1.0 / degi)


def _recip(dego, degi):
    return pl.pallas_call(
        _recip_body,
        out_shape=(jax.ShapeDtypeStruct((NP,), jnp.float32),
                   jax.ShapeDtypeStruct((NP,), jnp.float32)),
    )(dego, degi)


def _dense1_body(x_ref, w_ref, self_ref, po_ref, pi_ref):
    acc = jnp.dot(x_ref[...], w_ref[...], preferred_element_type=jnp.float32)
    self_ref[...] = acc[:, :S1]
    po_ref[...] = acc[:, S1:S1 + WP]
    pi_ref[...] = acc[:, S1 + WP:S1 + 2 * WP]


def _dense1(x, wcat):
    return pl.pallas_call(
        _dense1_body,
        grid=(NP // BR,),
        in_specs=[
            pl.BlockSpec((BR, D), lambda i: (i, 0)),
            pl.BlockSpec((D, S1 + 2 * WP), lambda i: (0, 0)),
        ],
        out_specs=[
            pl.BlockSpec((BR, S1), lambda i: (i, 0)),
            pl.BlockSpec((BR, WP), lambda i: (i, 0)),
            pl.BlockSpec((BR, WP), lambda i: (i, 0)),
        ],
        out_shape=[
            jax.ShapeDtypeStruct((NP, S1), jnp.float32),
            jax.ShapeDtypeStruct((NP, WP), jnp.float32),
            jax.ShapeDtypeStruct((NP, WP), jnp.float32),
        ],
    )(x, wcat)


def _edge_phase(src_hbm, dst_hbm, ew_hbm, rec_hbm, p_hbm, out_hbm, cid, tid,
                rec_v, idx_src, idx_dst, ew_v, norm_v, rows, acc):
    pltpu.sync_copy(rec_hbm, rec_v)
    _zero_rows_ref(rows, CK, WP)
    r0 = tid * RPT
    for k in range(RPT // CK):
        pltpu.sync_copy(rows, acc.at[pl.ds(r0 + k * CK, CK)])
    plsc.subcore_barrier()

    e_per_core = E // NC
    nchunks = e_per_core // CK
    trips = nchunks // NS + jnp.where(tid < (nchunks % NS), 1, 0)

    def body(k, _):
        c = tid + k * NS
        base = cid * e_per_core + c * CK
        pltpu.sync_copy(src_hbm.at[pl.ds(base, CK)], idx_src.at[0])
        pltpu.sync_copy(dst_hbm.at[pl.ds(base, CK)], idx_dst.at[0])
        pltpu.sync_copy(ew_hbm.at[pl.ds(base, CK)], ew_v)
        pltpu.sync_copy(p_hbm.at[idx_src.at[0]], rows)
        for g in range(CK // 16):
            sl = pl.ds(g * 16, 16)
            r16 = plsc.load_gather(rec_v, [idx_src[0, sl]])
            norm_v[sl] = ew_v[sl] * r16

        def scale(g, _):
            n16 = norm_v[pl.ds(g * 16, 16)]
            for l in range(16):
                s = n16[l]
                e = g * 16 + l
                for j in range(WP // 16):
                    sl = pl.ds(j * 16, 16)
                    rows[e, sl] = rows[e, sl] * s
            return 0

        lax.fori_loop(0, CK // 16, scale, 0)
        pltpu.sync_copy(rows, acc.at[idx_dst.at[0]], add=True)
        return 0

    lax.fori_loop(0, trips, body, 0)
    plsc.subcore_barrier()
    for k in range(RPT // CK):
        sl = pl.ds(r0 + k * CK, CK)
        pltpu.sync_copy(acc.at[sl], out_hbm.at[cid, sl])


@functools.partial(
    pl.kernel,
    out_type=(jax.ShapeDtypeStruct((NC, NP, WP), jnp.float32),
              jax.ShapeDtypeStruct((NC, NP, WP), jnp.float32)),
    mesh=_mesh,
    compiler_params=_CP,
    scratch_types=[
        pltpu.VMEM((NP,), jnp.float32),
        pltpu.VMEM((1, CK), jnp.int32),
        pltpu.VMEM((1, CK), jnp.int32),
        pltpu.VMEM((CK,), jnp.float32),
        pltpu.VMEM((CK,), jnp.float32),
        pltpu.VMEM((CK, WP), jnp.float32),
        pltpu.VMEM_SHARED((NP, WP), jnp.float32),
    ],
)
def _edge_kernel(row_hbm, col_hbm, ew_hbm, reco_hbm, reci_hbm, po_hbm, pi_hbm,
                 so_hbm, si_hbm,
                 rec_v, idx_src, idx_dst, ew_v, norm_v, rows, acc):
    cid = lax.axis_index("c")
    tid = lax.axis_index("s")
    _edge_phase(row_hbm, col_hbm, ew_hbm, reco_hbm, po_hbm, so_hbm, cid, tid,
                rec_v, idx_src, idx_dst, ew_v, norm_v, rows, acc)
    plsc.subcore_barrier()
    _edge_phase(col_hbm, row_hbm, ew_hbm, reci_hbm, pi_hbm, si_hbm, cid, tid,
                rec_v, idx_src, idx_dst, ew_v, norm_v, rows, acc)


def _gates1_body(self_ref, sop_ref, sip_ref, b_ref, w2_ref,
                 self2_ref, p2o_ref, p2i_ref):
    so = sop_ref[0] + sop_ref[1]
    si = sip_ref[0] + sip_ref[1]
    pre = self_ref[...][:, :2 * H1] + so[:, :2 * H1] + si[:, :2 * H1] + b_ref[...]
    z = jax.nn.sigmoid(pre[:, :H1])
    t = jnp.tanh(pre[:, H1:2 * H1])
    h1 = jax.nn.relu((1.0 - z) * t)
    acc = jnp.dot(h1, w2_ref[...], preferred_element_type=jnp.float32)
    self2_ref[...] = acc[:, :S2]
    p2o_ref[...] = acc[:, S2:S2 + WP]
    p2i_ref[...] = acc[:, S2 + WP:S2 + 2 * WP]


def _gates1(self1, sop, sip, bias1, w2cat):
    return pl.pallas_call(
        _gates1_body,
        grid=(NP // BR,),
        in_specs=[
            pl.BlockSpec((BR, S1), lambda i: (i, 0)),
            pl.BlockSpec((NC, BR, WP), lambda i: (0, i, 0)),
            pl.BlockSpec((NC, BR, WP), lambda i: (0, i, 0)),
            pl.BlockSpec((1, 2 * H1), lambda i: (0, 0)),
            pl.BlockSpec((H1, S2 + 2 * WP), lambda i: (0, 0)),
        ],
        out_specs=[
            pl.BlockSpec((BR, S2), lambda i: (i, 0)),
            pl.BlockSpec((BR, WP), lambda i: (i, 0)),
            pl.BlockSpec((BR, WP), lambda i: (i, 0)),
        ],
        out_shape=[
            jax.ShapeDtypeStruct((NP, S2), jnp.float32),
            jax.ShapeDtypeStruct((NP, WP), jnp.float32),
            jax.ShapeDtypeStruct((NP, WP), jnp.float32),
        ],
    )(self1, sop, sip, bias1, w2cat)


def _gates2_body(self_ref, sop_ref, sip_ref, b_ref, w_ref, out_ref):
    so = sop_ref[0] + sop_ref[1]
    si = sip_ref[0] + sip_ref[1]
    pre = self_ref[...][:, :2 * H2] + so[:, :2 * H2] + si[:, :2 * H2] + b_ref[...]
    z = jax.nn.sigmoid(pre[:, :H2])
    t = jnp.tanh(pre[:, H2:2 * H2])
    h2 = jax.nn.relu((1.0 - z) * t)
    w = w_ref[...]
    out_ref[...] = (jnp.sum(h2 * w[:1, :H2], axis=1, keepdims=True)
                    + w[1:2, :1])


def _gates2(self2, sop, sip, bias2, wlin):
    return pl.pallas_call(
        _gates2_body,
        grid=(NP // BR,),
        in_specs=[
            pl.BlockSpec((BR, S2), lambda i: (i, 0)),
            pl.BlockSpec((NC, BR, WP), lambda i: (0, i, 0)),
            pl.BlockSpec((NC, BR, WP), lambda i: (0, i, 0)),
            pl.BlockSpec((1, 2 * H2), lambda i: (0, 0)),
            pl.BlockSpec((2, H2), lambda i: (0, 0)),
        ],
        out_specs=pl.BlockSpec((BR, 1), lambda i: (i, 0)),
        out_shape=jax.ShapeDtypeStruct((NP, 1), jnp.float32),
    )(self2, sop, sip, bias2, wlin)


def kernel(x, edge_index, edge_weight, W1z, b1z, W1r, b1r, W1h, b1h,
           W2z, b2z, W2r, b2r, W2h, b2h, lin_W, lin_b):
    f32 = jnp.float32
    row = edge_index[0]
    col = edge_index[1]
    x_pad = jnp.zeros((NP, D), f32).at[:N].set(x)

    zp1 = jnp.zeros((D, WP - 2 * H1), f32)
    wcat1 = jnp.concatenate([
        W1z[0, 0, :D] + W1z[1, 0, :D], W1h[0, 0, :D] + W1h[1, 0, :D],
        jnp.zeros((D, S1 - 2 * H1), f32),
        W1z[0, 1, :D], W1h[0, 1, :D], zp1,
        W1z[1, 1, :D], W1h[1, 1, :D], zp1,
    ], axis=1)
    bias1 = jnp.concatenate([b1z, b1h]).reshape(1, 2 * H1)
    zp2 = jnp.zeros((H1, WP - 2 * H2), f32)
    wcat2 = jnp.concatenate([
        W2z[0, 0, :H1] + W2z[1, 0, :H1], W2h[0, 0, :H1] + W2h[1, 0, :H1],
        jnp.zeros((H1, S2 - 2 * H2), f32),
        W2z[0, 1, :H1], W2h[0, 1, :H1], zp2,
        W2z[1, 1, :H1], W2h[1, 1, :H1], zp2,
    ], axis=1)
    bias2 = jnp.concatenate([b2z, b2h]).reshape(1, 2 * H2)
    wlin = jnp.concatenate([
        lin_W.reshape(1, H2),
        jnp.concatenate([lin_b.reshape(1, 1),
                         jnp.zeros((1, H2 - 1), f32)], axis=1),
    ], axis=0)

    dego_f, degi_f = _deg_kernel(row, col, edge_weight)
    rec_o, rec_i = _recip(dego_f.reshape(NW, NP), degi_f.reshape(NW, NP))
    self1, p_o, p_i = _dense1(x_pad, wcat1)
    s_o, s_i = _edge_kernel(row, col, edge_weight, rec_o, rec_i, p_o, p_i)
    self2, p2o, p2i = _gates1(self1, s_o, s_i, bias1, wcat2)
    s2o, s2i = _edge_kernel(row, col, edge_weight, rec_o, rec_i, p2o, p2i)
    out = _gates2(self2, s2o, s2i, bias2, wlin)
    return out[:N]


# speedup vs baseline: 15.6207x; 1.4844x over previous
import functools

import jax
import jax.numpy as jnp
from jax import lax
from jax.experimental import pallas as pl
from jax.experimental.pallas import tpu as pltpu
from jax.experimental.pallas import tpu_sc as plsc

N = 10000
NP = 10240
E = 320000
D = 128
H1 = 50
H2 = 20
W1 = 112
W2 = 48
NC = 2
NS = 16
NW = NC * NS
RPT = NP // NS
CK = 128
NCHUNK = 2560
EP = NCHUNK * CK
IBLK = 40
NBLK = NCHUNK // IBLK
CPT = NCHUNK // NS
BPT = CPT // IBLK
BR = 1024

_mesh = plsc.VectorSubcoreMesh(core_axis_name="c", subcore_axis_name="s")
_CP = pltpu.CompilerParams(needs_layout_passes=False)
_CP_LIN = pltpu.CompilerParams(needs_layout_passes=False,
                               use_tc_tiling_on_sc=False)


def _zero_vec_ref(ref, n):
    def body(i, _):
        ref[pl.ds(i * 16, 16)] = jnp.zeros((16,), jnp.float32)
        return 0
    lax.fori_loop(0, n // 16, body, 0)


def _zero_rows_ref(ref, nrows, width):
    def body(i, _):
        for j in range(width // 16):
            ref[i, pl.ds(j * 16, 16)] = jnp.zeros((16,), jnp.float32)
        return 0
    lax.fori_loop(0, nrows, body, 0)


@functools.partial(
    pl.kernel,
    out_type=(jax.ShapeDtypeStruct((NW * NP,), jnp.float32),
              jax.ShapeDtypeStruct((NW * NP,), jnp.float32)),
    mesh=_mesh,
    compiler_params=_CP,
    scratch_types=[
        pltpu.VMEM((2 * IBLK, CK), jnp.int32),
        pltpu.VMEM((IBLK, CK), jnp.float32),
        pltpu.VMEM((NP,), jnp.float32),
        pltpu.VMEM((NP,), jnp.float32),
    ],
)
def _deg_kernel(idx_hbm, ew_hbm, out_o_hbm, out_i_hbm, idxb, ewb, acc_o, acc_i):
    cid = lax.axis_index("c")
    sid = lax.axis_index("s")
    wid = cid * NS + sid
    _zero_vec_ref(acc_o, NP)
    _zero_vec_ref(acc_i, NP)

    def block(b, _):
        blk = wid * (NBLK // NW) + b
        pltpu.sync_copy(idx_hbm.at[pl.ds(blk * 2 * IBLK, 2 * IBLK)], idxb)
        pltpu.sync_copy(ew_hbm.at[pl.ds(blk * IBLK, IBLK)], ewb)

        def chunk(k, _):
            for g in range(CK // 16):
                sl = pl.ds(g * 16, 16)
                w16 = ewb[k, sl]
                plsc.addupdate_scatter(acc_o, [idxb[k, sl]], w16)
                plsc.addupdate_scatter(acc_i, [idxb[IBLK + k, sl]], w16)
            return 0

        lax.fori_loop(0, IBLK, chunk, 0)
        return 0

    lax.fori_loop(0, NBLK // NW, block, 0)
    pltpu.sync_copy(acc_o, out_o_hbm.at[pl.ds(wid * NP, NP)])
    pltpu.sync_copy(acc_i, out_i_hbm.at[pl.ds(wid * NP, NP)])


def _dense1_body(x_ref, w_ref, dego_ref, degi_ref,
                 self_ref, po_ref, pi_ref, reco_ref, reci_ref):
    acc = jnp.dot(x_ref[...], w_ref[...], preferred_element_type=jnp.float32,
                  precision=lax.Precision.HIGHEST)
    dego = jnp.sum(dego_ref[...], axis=0)
    degi = jnp.sum(degi_ref[...], axis=0)
    reco = jnp.where(dego == 0.0, 1.0, 1.0 / dego)
    reci = jnp.where(degi == 0.0, 1.0, 1.0 / degi)
    reco_ref[...] = reco
    reci_ref[...] = reci
    self_ref[...] = acc[:, :W1]
    po_ref[...] = acc[:, W1:2 * W1] * reco[:, None]
    pi_ref[...] = acc[:, 2 * W1:3 * W1] * reci[:, None]


def _dense1(x, wcat, dego, degi):
    return pl.pallas_call(
        _dense1_body,
        grid=(NP // BR,),
        in_specs=[
            pl.BlockSpec((BR, D), lambda i: (i, 0)),
            pl.BlockSpec((D, 3 * W1), lambda i: (0, 0)),
            pl.BlockSpec((NW, BR), lambda i: (0, i)),
            pl.BlockSpec((NW, BR), lambda i: (0, i)),
        ],
        out_specs=[
            pl.BlockSpec((BR, W1), lambda i: (i, 0)),
            pl.BlockSpec((BR, W1), lambda i: (i, 0)),
            pl.BlockSpec((BR, W1), lambda i: (i, 0)),
            pl.BlockSpec((BR,), lambda i: (i,)),
            pl.BlockSpec((BR,), lambda i: (i,)),
        ],
        out_shape=[
            jax.ShapeDtypeStruct((NP, W1), jnp.float32),
            jax.ShapeDtypeStruct((NP, W1), jnp.float32),
            jax.ShapeDtypeStruct((NP, W1), jnp.float32),
            jax.ShapeDtypeStruct((NP,), jnp.float32),
            jax.ShapeDtypeStruct((NP,), jnp.float32),
        ],
    )(x, wcat, dego, degi)


def _make_edge_kernel(width):
    def _edge_dir(idx_hbm, ew_hbm, p_hbm, out_hbm, tid,
                  idxb, ewb, rows0, rows1, sem0, sem1, acc, src_half):
        s_base = src_half * IBLK
        d_base = (1 - src_half) * IBLK
        _zero_rows_ref(rows0, CK, width)
        r0 = tid * RPT
        for k in range(RPT // CK):
            pltpu.sync_copy(rows0, acc.at[pl.ds(r0 + k * CK, CK)])
        plsc.subcore_barrier()

        def gather(c, rows, sem):
            pltpu.make_async_copy(
                p_hbm.at[idxb.at[s_base + c]], rows, sem).start()

        def finish(c, rows, sem):
            pltpu.make_async_copy(
                p_hbm.at[idxb.at[s_base + c]], rows, sem).wait()

            def scale(g, _):
                sl = pl.ds(g * 16, 16)
                w16 = ewb[c, sl]
                for l in range(16):
                    s = w16[l]
                    e16 = g * 16 + l
                    for j in range(width // 16):
                        slj = pl.ds(j * 16, 16)
                        rows[e16, slj] = rows[e16, slj] * s
                return 0

            lax.fori_loop(0, CK // 16, scale, 0)
            pltpu.sync_copy(rows, acc.at[idxb.at[d_base + c]], add=True)

        def block(b, _):
            blk = tid * BPT + b
            pltpu.sync_copy(idx_hbm.at[pl.ds(blk * 2 * IBLK, 2 * IBLK)], idxb)
            pltpu.sync_copy(ew_hbm.at[pl.ds(blk * IBLK, IBLK)], ewb)
            gather(0, rows0, sem0)

            def pair(p, _):
                gather(2 * p + 1, rows1, sem1)
                finish(2 * p, rows0, sem0)

                @pl.when(p < IBLK // 2 - 1)
                def _():
                    gather(2 * p + 2, rows0, sem0)

                finish(2 * p + 1, rows1, sem1)
                return 0

            lax.fori_loop(0, IBLK // 2, pair, 0)
            return 0

        lax.fori_loop(0, BPT, block, 0)
        plsc.subcore_barrier()
        for k in range(RPT // CK):
            sl = pl.ds(r0 + k * CK, CK)
            pltpu.sync_copy(acc.at[sl], out_hbm.at[sl])

    @functools.partial(
        pl.kernel,
        out_type=(jax.ShapeDtypeStruct((NP, width), jnp.float32),
                  jax.ShapeDtypeStruct((NP, width), jnp.float32)),
        mesh=_mesh,
        compiler_params=_CP_LIN,
        scratch_types=[
            pltpu.VMEM((2 * IBLK, CK), jnp.int32),
            pltpu.VMEM((IBLK, CK), jnp.float32),
            pltpu.VMEM((CK, width), jnp.float32),
            pltpu.VMEM((CK, width), jnp.float32),
            pltpu.SemaphoreType.DMA,
            pltpu.SemaphoreType.DMA,
            pltpu.VMEM_SHARED((NP, width), jnp.float32),
        ],
    )
    def _edge_kernel(idx_hbm, ew_hbm, po_hbm, pi_hbm, so_hbm, si_hbm,
                     idxb, ewb, rows0, rows1, sem0, sem1, acc):
        cid = lax.axis_index("c")
        tid = lax.axis_index("s")

        @pl.when(cid == 0)
        def _():
            _edge_dir(idx_hbm, ew_hbm, po_hbm, so_hbm, tid,
                      idxb, ewb, rows0, rows1, sem0, sem1, acc, src_half=0)

        @pl.when(cid == 1)
        def _():
            _edge_dir(idx_hbm, ew_hbm, pi_hbm, si_hbm, tid,
                      idxb, ewb, rows0, rows1, sem0, sem1, acc, src_half=1)

    return _edge_kernel


_edge_kernel_1 = _make_edge_kernel(W1)
_edge_kernel_2 = _make_edge_kernel(W2)


def _gates1_body(self_ref, so_ref, si_ref, b_ref, w2_ref, reco_ref, reci_ref,
                 self2_ref, p2o_ref, p2i_ref):
    pre = (self_ref[...][:, :2 * H1] + so_ref[...][:, :2 * H1]
           + si_ref[...][:, :2 * H1] + b_ref[...])
    z = jax.nn.sigmoid(pre[:, :H1])
    t = jnp.tanh(pre[:, H1:2 * H1])
    h1 = jax.nn.relu((1.0 - z) * t)
    acc = jnp.dot(h1, w2_ref[...], preferred_element_type=jnp.float32,
                  precision=lax.Precision.HIGHEST)
    self2_ref[...] = acc[:, :W2]
    p2o_ref[...] = acc[:, W2:2 * W2] * reco_ref[...][:, None]
    p2i_ref[...] = acc[:, 2 * W2:3 * W2] * reci_ref[...][:, None]


def _gates1(self1, so, si, bias1, w2cat, reco, reci):
    return pl.pallas_call(
        _gates1_body,
        grid=(NP // BR,),
        in_specs=[
            pl.BlockSpec((BR, W1), lambda i: (i, 0)),
            pl.BlockSpec((BR, W1), lambda i: (i, 0)),
            pl.BlockSpec((BR, W1), lambda i: (i, 0)),
            pl.BlockSpec((1, 2 * H1), lambda i: (0, 0)),
            pl.BlockSpec((H1, 3 * W2), lambda i: (0, 0)),
            pl.BlockSpec((BR,), lambda i: (i,)),
            pl.BlockSpec((BR,), lambda i: (i,)),
        ],
        out_specs=[
            pl.BlockSpec((BR, W2), lambda i: (i, 0)),
            pl.BlockSpec((BR, W2), lambda i: (i, 0)),
            pl.BlockSpec((BR, W2), lambda i: (i, 0)),
        ],
        out_shape=[
            jax.ShapeDtypeStruct((NP, W2), jnp.float32),
            jax.ShapeDtypeStruct((NP, W2), jnp.float32),
            jax.ShapeDtypeStruct((NP, W2), jnp.float32),
        ],
    )(self1, so, si, bias1, w2cat, reco, reci)


def _gates2_body(self_ref, so_ref, si_ref, b_ref, w_ref, out_ref):
    pre = (self_ref[...][:, :2 * H2] + so_ref[...][:, :2 * H2]
           + si_ref[...][:, :2 * H2] + b_ref[...])
    z = jax.nn.sigmoid(pre[:, :H2])
    t = jnp.tanh(pre[:, H2:2 * H2])
    h2 = jax.nn.relu((1.0 - z) * t)
    w = w_ref[...]
    out_ref[...] = (jnp.sum(h2 * w[:1, :H2], axis=1, keepdims=True)
                    + w[1:2, :1])


def _gates2(self2, so, si, bias2, wlin):
    return pl.pallas_call(
        _gates2_body,
        grid=(NP // BR,),
        in_specs=[
            pl.BlockSpec((BR, W2), lambda i: (i, 0)),
            pl.BlockSpec((BR, W2), lambda i: (i, 0)),
            pl.BlockSpec((BR, W2), lambda i: (i, 0)),
            pl.BlockSpec((1, 2 * H2), lambda i: (0, 0)),
            pl.BlockSpec((2, H2), lambda i: (0, 0)),
        ],
        out_specs=pl.BlockSpec((BR, 1), lambda i: (i, 0)),
        out_shape=jax.ShapeDtypeStruct((NP, 1), jnp.float32),
    )(self2, so, si, bias2, wlin)


def kernel(x, edge_index, edge_weight, W1z, b1z, W1r, b1r, W1h, b1h,
           W2z, b2z, W2r, b2r, W2h, b2h, lin_W, lin_b):
    f32 = jnp.float32
    row2d = jnp.concatenate(
        [edge_index[0], jnp.zeros((EP - E,), jnp.int32)]).reshape(NBLK, IBLK, CK)
    col2d = jnp.concatenate(
        [edge_index[1], jnp.zeros((EP - E,), jnp.int32)]).reshape(NBLK, IBLK, CK)
    idx2d = jnp.concatenate([row2d, col2d], axis=1).reshape(NBLK * 2 * IBLK, CK)
    ew2d = jnp.concatenate(
        [edge_weight, jnp.zeros((EP - E,), f32)]).reshape(NCHUNK, CK)
    x_pad = jnp.zeros((NP, D), f32).at[:N].set(x)

    zs1 = jnp.zeros((D, W1 - 2 * H1), f32)
    wcat1 = jnp.concatenate([
        W1z[0, 0, :D] + W1z[1, 0, :D], W1h[0, 0, :D] + W1h[1, 0, :D], zs1,
        W1z[0, 1, :D], W1h[0, 1, :D], zs1,
        W1z[1, 1, :D], W1h[1, 1, :D], zs1,
    ], axis=1)
    bias1 = jnp.concatenate([b1z, b1h]).reshape(1, 2 * H1)
    zs2 = jnp.zeros((H1, W2 - 2 * H2), f32)
    wcat2 = jnp.concatenate([
        W2z[0, 0, :H1] + W2z[1, 0, :H1], W2h[0, 0, :H1] + W2h[1, 0, :H1], zs2,
        W2z[0, 1, :H1], W2h[0, 1, :H1], zs2,
        W2z[1, 1, :H1], W2h[1, 1, :H1], zs2,
    ], axis=1)
    bias2 = jnp.concatenate([b2z, b2h]).reshape(1, 2 * H2)
    wlin = jnp.concatenate([
        lin_W.reshape(1, H2),
        jnp.concatenate([lin_b.reshape(1, 1),
                         jnp.zeros((1, H2 - 1), f32)], axis=1),
    ], axis=0)

    dego_f, degi_f = _deg_kernel(idx2d, ew2d)
    self1, p_o, p_i, rec_o, rec_i = _dense1(
        x_pad, wcat1, dego_f.reshape(NW, NP), degi_f.reshape(NW, NP))
    s_o, s_i = _edge_kernel_1(idx2d, ew2d, p_o, p_i)
    self2, p2o, p2i = _gates1(self1, s_o, s_i, bias1, wcat2, rec_o, rec_i)
    s2o, s2i = _edge_kernel_2(idx2d, ew2d, p2o, p2i)
    out = _gates2(self2, s2o, s2i, bias2, wlin)
    return out[:N]
